# baseline (speedup 1.0000x reference)
import functools

import numpy as np
import jax
import jax.numpy as jnp
from jax import lax
from jax.experimental import pallas as pl
from jax.experimental.pallas import tpu as pltpu
from jax.experimental.pallas import tpu_sc as plsc

N_NODES = 10000
D = 128
NQ = 4
DQ = D // NQ

NC = 2
NS = 16
CHUNK = 64
CHUNKS_PER_TILE = 320
E_PER_TILE = CHUNK * CHUNKS_PER_TILE
E_PAD = NS * E_PER_TILE
ACC_ROWS = 10240
STRIPE = ACC_ROWS // NS
NB = 4
NSB = 2

_QIDX = np.concatenate([np.arange(0, DQ, 2), np.arange(1, DQ, 2)])


def _sc_aggregate(sd_p, w_p, x_bits):
    mesh = plsc.VectorSubcoreMesh(core_axis_name="c", subcore_axis_name="s")

    @functools.partial(
        pl.kernel,
        out_type=jax.ShapeDtypeStruct((NC, 2, ACC_ROWS, DQ), jnp.float32),
        mesh=mesh,
        scratch_types=[
            pltpu.VMEM((CHUNKS_PER_TILE, CHUNK), jnp.int32),
            pltpu.VMEM((CHUNKS_PER_TILE, CHUNK), jnp.int32),
            pltpu.VMEM((CHUNKS_PER_TILE, CHUNK), jnp.float32),
            pltpu.VMEM((CHUNKS_PER_TILE, CHUNK), jnp.int32),
            pltpu.VMEM((NB, CHUNK, DQ), jnp.bfloat16),
            pltpu.VMEM((NSB, CHUNK, DQ), jnp.float32),
            pltpu.VMEM((CHUNK, DQ), jnp.float32),
            pltpu.VMEM_SHARED((ACC_ROWS, DQ), jnp.bfloat16),
            pltpu.VMEM_SHARED((ACC_ROWS, DQ), jnp.float32),
        ] + [pltpu.SemaphoreType.DMA] * (NB + NSB),
        compiler_params=pltpu.CompilerParams(use_tc_tiling_on_sc=False,
                                             needs_layout_passes=False),
    )
    def agg(sd_hbm, w_hbm, xbflat_hbm, out_hbm,
            src_v, dst_v, w_v, srco_v, rows_i, rows_f, zero_v, x_sh, acc_sh,
            *sems):
        gsem = sems[:NB]
        ssem = sems[NB:]
        cid = lax.axis_index("c")
        sid = lax.axis_index("s")

        pltpu.sync_copy(sd_hbm.at[sid], src_v)
        pltpu.sync_copy(w_hbm.at[sid], w_v)

        def unpack_idx(j, carry):
            for k in range(CHUNK // 16):
                p = src_v[j, pl.ds(k * 16, 16)]
                dst_v[j, pl.ds(k * 16, 16)] = p >> 16
                src_v[j, pl.ds(k * 16, 16)] = p & jnp.int32(0xFFFF)
            return carry
        lax.fori_loop(0, CHUNKS_PER_TILE, unpack_idx, 0)

        def zero_row(i, carry):
            for k in range(DQ // 16):
                zero_v[i, pl.ds(k * 16, 16)] = jnp.zeros((16,), jnp.float32)
            return carry
        lax.fori_loop(0, CHUNK, zero_row, 0)

        def run_quarter(q):
            qbase = (cid * 2 + q) * ACC_ROWS

            def offset_idx(jj, carry):
                for k in range(CHUNK // 16):
                    srco_v[jj, pl.ds(k * 16, 16)] = (
                        src_v[jj, pl.ds(k * 16, 16)] + qbase)
                return carry
            lax.fori_loop(0, CHUNKS_PER_TILE, offset_idx, 0)

            pltpu.sync_copy(
                xbflat_hbm.at[pl.ds(qbase + sid * STRIPE, STRIPE)],
                x_sh.at[pl.ds(sid * STRIPE, STRIPE)])
            for b in range(STRIPE // CHUNK):
                pltpu.sync_copy(
                    zero_v, acc_sh.at[pl.ds(sid * STRIPE + b * CHUNK, CHUNK)])
            plsc.subcore_barrier()

            for i in range(NB - 1):
                if i % 2 == 0:
                    pltpu.async_copy(x_sh.at[src_v.at[i]], rows_i.at[i],
                                     gsem[i])
                else:
                    pltpu.async_copy(xbflat_hbm.at[srco_v.at[i]],
                                     rows_i.at[i], gsem[i])
            for i in range(NSB):
                pltpu.async_copy(zero_v, acc_sh.at[dst_v.at[0]], ssem[i],
                                 add=True)

            def outer(j0, carry):
                for b in range(NB):
                    j = j0 * NB + b
                    sb = b % NSB
                    pltpu.make_async_copy(
                        x_sh.at[src_v.at[0]], rows_i.at[b], gsem[b]).wait()
                    pltpu.make_async_copy(
                        rows_f.at[sb], acc_sh.at[dst_v.at[0]], ssem[sb]).wait()
                    for g in range(CHUNK // 16):
                        wv = w_v[j, pl.ds(g * 16, 16)]
                        for l in range(16):
                            e = g * 16 + l
                            we = wv[l]
                            xw = rows_i[b, e, :]
                            lo, hi = plsc.unpack(
                                xw, format=plsc.PackFormat.INTERLEAVED)
                            rows_f[sb, e, pl.ds(0, 16)] = lo * we
                            rows_f[sb, e, pl.ds(16, 16)] = hi * we
                    pltpu.async_copy(rows_f.at[sb], acc_sh.at[dst_v.at[j]],
                                     ssem[sb], add=True)
                    pb = (b + NB - 1) % NB

                    @pl.when(j + NB - 1 < CHUNKS_PER_TILE)
                    def _():
                        if (b + NB - 1) % 2 == 0:
                            pltpu.async_copy(x_sh.at[src_v.at[j + NB - 1]],
                                             rows_i.at[pb], gsem[pb])
                        else:
                            pltpu.async_copy(
                                xbflat_hbm.at[srco_v.at[j + NB - 1]],
                                rows_i.at[pb], gsem[pb])
                return carry
            lax.fori_loop(0, CHUNKS_PER_TILE // NB, outer, 0)

            for i in range(NSB):
                pltpu.make_async_copy(
                    rows_f.at[i], acc_sh.at[dst_v.at[0]], ssem[i]).wait()
            plsc.subcore_barrier()
            pltpu.sync_copy(acc_sh.at[pl.ds(sid * STRIPE, STRIPE)],
                            out_hbm.at[cid, q, pl.ds(sid * STRIPE, STRIPE)])

        run_quarter(0)
        plsc.subcore_barrier()
        run_quarter(1)

    return agg(sd_p, w_p, x_bits.reshape(NQ * ACC_ROWS, DQ))


def _tc_finish(partials, wq):
    def body(p_ref, w_ref, o_ref):
        dn = (((1,), (1,)), ((), ()))
        acc = lax.dot_general(p_ref[0, 0, :N_NODES, :], w_ref[0], dn,
                              preferred_element_type=jnp.float32)
        for i, (c, q) in enumerate([(0, 1), (1, 0), (1, 1)]):
            acc = acc + lax.dot_general(
                p_ref[c, q, :N_NODES, :], w_ref[i + 1], dn,
                preferred_element_type=jnp.float32)
        o_ref[...] = acc

    return pl.pallas_call(
        body,
        out_shape=jax.ShapeDtypeStruct((N_NODES, D), jnp.float32),
    )(partials, wq)


@jax.jit
def kernel(edge_index, edge_weight, X, W):
    n_edges = edge_index.shape[1]
    pad = E_PAD - n_edges
    src = jnp.concatenate([edge_index[1].astype(jnp.int32),
                           jnp.zeros((pad,), jnp.int32)])
    dst = jnp.concatenate([edge_index[0].astype(jnp.int32),
                           jnp.zeros((pad,), jnp.int32)])
    w = jnp.concatenate([edge_weight, jnp.zeros((pad,), jnp.float32)])
    sd_p = (src | (dst << 16)).reshape(NS, CHUNKS_PER_TILE, CHUNK)
    w_p = w.reshape(NS, CHUNKS_PER_TILE, CHUNK)

    x_pad = jnp.concatenate(
        [X, jnp.zeros((ACC_ROWS - N_NODES, D), X.dtype)]).astype(jnp.bfloat16)
    x_bits = x_pad.reshape(ACC_ROWS, NQ, DQ).transpose(1, 0, 2)
    x_bits = x_bits.reshape(NC, 2, ACC_ROWS, DQ)

    wq = jnp.stack([W[:, q * DQ + _QIDX] for q in range(NQ)])

    partials = _sc_aggregate(sd_p, w_p, x_bits)
    return _tc_finish(partials, wq)


# speedup vs baseline: 1.1611x; 1.1611x over previous
import functools

import numpy as np
import jax
import jax.numpy as jnp
from jax import lax
from jax.experimental import pallas as pl
from jax.experimental.pallas import tpu as pltpu
from jax.experimental.pallas import tpu_sc as plsc

N_NODES = 10000
D = 128
NQ = 4
DQ = D // NQ

NC = 2
NS = 16
CHUNK = 128
CHUNKS_PER_TILE = 160
E_PER_TILE = CHUNK * CHUNKS_PER_TILE
E_PAD = NS * E_PER_TILE
ACC_ROWS = 10240
STRIPE = ACC_ROWS // NS
NB = 4
NSB = 2

_QIDX = np.concatenate([np.arange(0, DQ, 2), np.arange(1, DQ, 2)])


def _sc_aggregate(sd_p, w_p, x_bits):
    mesh = plsc.VectorSubcoreMesh(core_axis_name="c", subcore_axis_name="s")

    @functools.partial(
        pl.kernel,
        out_type=jax.ShapeDtypeStruct((NC, 2, ACC_ROWS, DQ), jnp.float32),
        mesh=mesh,
        scratch_types=[
            pltpu.VMEM((CHUNKS_PER_TILE, CHUNK), jnp.int32),
            pltpu.VMEM((CHUNKS_PER_TILE, CHUNK), jnp.int32),
            pltpu.VMEM((CHUNKS_PER_TILE, CHUNK), jnp.float32),
            pltpu.VMEM((NB, CHUNK, DQ), jnp.bfloat16),
            pltpu.VMEM((NSB, CHUNK, DQ), jnp.float32),
            pltpu.VMEM((128, DQ), jnp.float32),
            pltpu.VMEM_SHARED((ACC_ROWS, DQ), jnp.bfloat16),
            pltpu.VMEM_SHARED((ACC_ROWS, DQ), jnp.float32),
        ] + [pltpu.SemaphoreType.DMA] * (NB + NSB),
        compiler_params=pltpu.CompilerParams(use_tc_tiling_on_sc=False,
                                             needs_layout_passes=False),
    )
    def agg(sd_hbm, w_hbm, xb_hbm, out_hbm,
            src_v, dst_v, w_v, rows_i, rows_f, zero_v, x_sh, acc_sh, *sems):
        gsem = sems[:NB]
        ssem = sems[NB:]
        cid = lax.axis_index("c")
        sid = lax.axis_index("s")

        pltpu.sync_copy(sd_hbm.at[sid], src_v)
        pltpu.sync_copy(w_hbm.at[sid], w_v)

        def unpack_idx(j, carry):
            for k in range(CHUNK // 16):
                p = src_v[j, pl.ds(k * 16, 16)]
                dst_v[j, pl.ds(k * 16, 16)] = p >> 16
                src_v[j, pl.ds(k * 16, 16)] = p & jnp.int32(0xFFFF)
            return carry
        lax.fori_loop(0, CHUNKS_PER_TILE, unpack_idx, 0)

        def zero_row(i, carry):
            for k in range(DQ // 16):
                zero_v[i, pl.ds(k * 16, 16)] = jnp.zeros((16,), jnp.float32)
            return carry
        lax.fori_loop(0, 128, zero_row, 0)

        def run_quarter(q):
            pltpu.sync_copy(xb_hbm.at[cid, q, pl.ds(sid * STRIPE, STRIPE)],
                            x_sh.at[pl.ds(sid * STRIPE, STRIPE)])
            for b in range(STRIPE // 128):
                pltpu.sync_copy(zero_v,
                                acc_sh.at[pl.ds(sid * STRIPE + b * 128, 128)])
            plsc.subcore_barrier()

            for i in range(NB - 1):
                pltpu.async_copy(x_sh.at[src_v.at[i]], rows_i.at[i], gsem[i])
            for i in range(NSB):
                pltpu.async_copy(zero_v, acc_sh.at[dst_v.at[0]], ssem[i],
                                 add=True)

            def outer(j0, carry):
                for b in range(NB):
                    j = j0 * NB + b
                    sb = b % NSB
                    pltpu.make_async_copy(
                        x_sh.at[src_v.at[0]], rows_i.at[b], gsem[b]).wait()
                    pltpu.make_async_copy(
                        rows_f.at[sb], acc_sh.at[dst_v.at[0]], ssem[sb]).wait()
                    for g in range(CHUNK // 16):
                        wv = w_v[j, pl.ds(g * 16, 16)]
                        for l in range(16):
                            e = g * 16 + l
                            we = wv[l]
                            xw = rows_i[b, e, :]
                            lo, hi = plsc.unpack(
                                xw, format=plsc.PackFormat.INTERLEAVED)
                            rows_f[sb, e, pl.ds(0, 16)] = lo * we
                            rows_f[sb, e, pl.ds(16, 16)] = hi * we
                    pltpu.async_copy(rows_f.at[sb], acc_sh.at[dst_v.at[j]],
                                     ssem[sb], add=True)
                    pb = (b + NB - 1) % NB

                    @pl.when(j + NB - 1 < CHUNKS_PER_TILE)
                    def _():
                        pltpu.async_copy(x_sh.at[src_v.at[j + NB - 1]],
                                         rows_i.at[pb], gsem[pb])
                return carry
            lax.fori_loop(0, CHUNKS_PER_TILE // NB, outer, 0)

            for i in range(NSB):
                pltpu.make_async_copy(
                    rows_f.at[i], acc_sh.at[dst_v.at[0]], ssem[i]).wait()
            plsc.subcore_barrier()
            pltpu.sync_copy(acc_sh.at[pl.ds(sid * STRIPE, STRIPE)],
                            out_hbm.at[cid, q, pl.ds(sid * STRIPE, STRIPE)])

        run_quarter(0)
        plsc.subcore_barrier()
        run_quarter(1)

    return agg(sd_p, w_p, x_bits)


def _tc_finish(partials, wq):
    def body(p_ref, w_ref, o_ref):
        dn = (((1,), (1,)), ((), ()))
        acc = lax.dot_general(p_ref[0, 0, :N_NODES, :], w_ref[0], dn,
                              preferred_element_type=jnp.float32)
        for i, (c, q) in enumerate([(0, 1), (1, 0), (1, 1)]):
            acc = acc + lax.dot_general(
                p_ref[c, q, :N_NODES, :], w_ref[i + 1], dn,
                preferred_element_type=jnp.float32)
        o_ref[...] = acc

    return pl.pallas_call(
        body,
        out_shape=jax.ShapeDtypeStruct((N_NODES, D), jnp.float32),
    )(partials, wq)


@jax.jit
def kernel(edge_index, edge_weight, X, W):
    n_edges = edge_index.shape[1]
    pad = E_PAD - n_edges
    src = jnp.concatenate([edge_index[1].astype(jnp.int32),
                           jnp.zeros((pad,), jnp.int32)])
    dst = jnp.concatenate([edge_index[0].astype(jnp.int32),
                           jnp.zeros((pad,), jnp.int32)])
    w = jnp.concatenate([edge_weight, jnp.zeros((pad,), jnp.float32)])
    sd_p = (src | (dst << 16)).reshape(NS, CHUNKS_PER_TILE, CHUNK)
    w_p = w.reshape(NS, CHUNKS_PER_TILE, CHUNK)

    x_pad = jnp.concatenate(
        [X, jnp.zeros((ACC_ROWS - N_NODES, D), X.dtype)]).astype(jnp.bfloat16)
    x_bits = x_pad.reshape(ACC_ROWS, NQ, DQ).transpose(1, 0, 2)
    x_bits = x_bits.reshape(NC, 2, ACC_ROWS, DQ)

    wq = jnp.stack([W[:, q * DQ + _QIDX] for q in range(NQ)])

    partials = _sc_aggregate(sd_p, w_p, x_bits)
    return _tc_finish(partials, wq)
